# baseline (speedup 1.0000x reference)
import functools

import jax
import jax.numpy as jnp
from jax import lax
from jax.experimental import pallas as pl
from jax.experimental.pallas import tpu as pltpu
from jax.experimental.pallas import tpu_sc as plsc

N = 10000
E = 160000
EPAD = 163840
NSUB = 16
NW = 32
PT_E = EPAD // NW
NBKT = 8
BSZ = 1256
JUNK = 1256
ACC = 1264
CAP = 1024
CAPR = CAP // 128
BKT_ROWS = NW * CAPR
BN = 1000

@functools.lru_cache(maxsize=None)
def _mesh():
    return plsc.VectorSubcoreMesh(core_axis_name="c", subcore_axis_name="s",
                                  num_cores=2, num_subcores=NSUB)


SEG = CAP + 16


@functools.lru_cache(maxsize=None)
def _partition():
    out_type = (
        jax.ShapeDtypeStruct((NBKT * NW * CAP,), jnp.int32),
        jax.ShapeDtypeStruct((NBKT * NW * CAP,), jnp.int32),
    )
    scratch = [
        pltpu.VMEM((PT_E,), jnp.int32),
        pltpu.VMEM((PT_E,), jnp.int32),
        pltpu.VMEM((NBKT * SEG,), jnp.int32),
        pltpu.VMEM((NBKT * SEG,), jnp.int32),
        pltpu.SMEM((1, NBKT), jnp.int32),
    ]

    def body(src_r, dst_r, bsrc, bdst, esrc, edst, sbig, dbig, offsm):
        c = lax.axis_index("c")
        s = lax.axis_index("s")
        w = c * NSUB + s

        eoff = pl.multiple_of(w * PT_E, 8)
        pltpu.sync_copy(src_r.at[pl.ds(eoff, PT_E)], esrc)
        pltpu.sync_copy(dst_r.at[pl.ds(eoff, PT_E)], edst)

        zi = jnp.zeros((16,), jnp.int32)
        ji = jnp.full((16,), JUNK, jnp.int32)

        def pf(i, cy):
            sbig[pl.ds(i * 16, 16)] = zi
            dbig[pl.ds(i * 16, 16)] = ji
            return cy

        lax.fori_loop(0, NBKT * SEG // 16, pf, 0)

        for b in range(NBKT):
            offsm[0, b] = 0

        bszv = jnp.full((16,), BSZ, jnp.int32)
        iota = lax.iota(jnp.int32, 16)

        def step(k, cy):
            svv = esrc[pl.ds(k * 16, 16)]
            dvv = edst[pl.ds(k * 16, 16)]
            bvv = lax.div(dvv, bszv)
            lvv = dvv - bvv * bszv
            for t in range(16):
                b = bvv[t]
                off = offsm[0, b]
                lane = off & 15
                base = b * SEG + (off - lane)
                lm = iota == lane
                vs = sbig[pl.ds(base, 16)]
                sbig[pl.ds(base, 16)] = jnp.where(
                    lm, jnp.full((16,), svv[t], jnp.int32), vs)
                vd = dbig[pl.ds(base, 16)]
                dbig[pl.ds(base, 16)] = jnp.where(
                    lm, jnp.full((16,), lvv[t], jnp.int32), vd)
                offsm[0, b] = jnp.minimum(off + 1, CAP)
            return cy

        lax.fori_loop(0, PT_E // 16, step, 0)

        for b in range(NBKT):
            boff = pl.multiple_of(b * NW * CAP + w * CAP, 8)
            pltpu.sync_copy(sbig.at[pl.ds(b * SEG, CAP)],
                            bsrc.at[pl.ds(boff, CAP)])
            pltpu.sync_copy(dbig.at[pl.ds(b * SEG, CAP)],
                            bdst.at[pl.ds(boff, CAP)])

    return pl.kernel(body, out_type=out_type, mesh=_mesh(),
                     scratch_types=scratch, name="gin_partition")


@functools.lru_cache(maxsize=None)
def _segsum(d):
    sl = d // 128
    out_type = jax.ShapeDtypeStruct((N, sl, 128), jnp.float32)
    scratch = [
        pltpu.VMEM((CAPR, 128), jnp.int32),
        pltpu.VMEM((CAPR, 128), jnp.int32),
        pltpu.VMEM((128, sl, 128), jnp.float32),
        pltpu.VMEM_SHARED((ACC, sl, 128), jnp.float32),
        pltpu.SemaphoreType.DMA,
    ]

    def body(h_r, bsrc, bdst, zeros_r, agg, sidx, didx, gbuf, accum, gsem):
        c = lax.axis_index("c")
        s = lax.axis_index("s")

        def run_bucket(b):
            zoff = pl.multiple_of(s * 80, 8)

            @pl.when(s < NSUB - 1)
            def _():
                pltpu.sync_copy(zeros_r.at[pl.ds(zoff, 80)],
                                accum.at[pl.ds(zoff, 80)])

            @pl.when(s == NSUB - 1)
            def _():
                pltpu.sync_copy(zeros_r.at[pl.ds(1200, ACC - 1200)],
                                accum.at[pl.ds(1200, ACC - 1200)])

            plsc.subcore_barrier()

            for halfw in range(2):
                w2 = 2 * s + halfw
                roff = pl.multiple_of(w2 * CAPR, 8)
                pltpu.sync_copy(bsrc.at[b].at[pl.ds(roff, CAPR)], sidx)
                pltpu.sync_copy(bdst.at[b].at[pl.ds(roff, CAPR)], didx)

                for r in range(CAPR):
                    pltpu.async_copy(h_r.at[sidx.at[r]], gbuf, gsem).wait()
                    pltpu.sync_copy(gbuf, accum.at[didx.at[r]], add=True)

            plsc.subcore_barrier()
            ooff = pl.multiple_of(b * BSZ + s * 80, 8)
            nlast = (BSZ if b < NBKT - 1 else N - (NBKT - 1) * BSZ) - 1200

            @pl.when(s < NSUB - 1)
            def _():
                pltpu.sync_copy(accum.at[pl.ds(zoff, 80)],
                                agg.at[pl.ds(ooff, 80)])

            @pl.when(s == NSUB - 1)
            def _():
                pltpu.sync_copy(accum.at[pl.ds(1200, nlast)],
                                agg.at[pl.ds(pl.multiple_of(
                                    b * BSZ + 1200, 8), nlast)])

            plsc.subcore_barrier()

        for bi in range(NBKT // 2):
            @pl.when(c == 0)
            def _(_b=bi):
                run_bucket(_b)

            @pl.when(c == 1)
            def _(_b=NBKT // 2 + bi):
                run_bucket(_b)

    return pl.kernel(body, out_type=out_type, mesh=_mesh(),
                     scratch_types=scratch, name=f"gin_segsum{d}")


def _ln_relu_mlp(z, w1, b1, w2, b2, g, bt):
    t = jnp.maximum(jnp.dot(z, w1, preferred_element_type=jnp.float32) + b1, 0.0)
    t = jnp.dot(t, w2, preferred_element_type=jnp.float32) + b2
    mu = jnp.mean(t, axis=-1, keepdims=True)
    d = t - mu
    var = jnp.mean(d * d, axis=-1, keepdims=True)
    t = d * lax.rsqrt(var + 1e-5) * g + bt
    return jnp.maximum(t, 0.0)


@functools.lru_cache(maxsize=None)
def _mlp_hidden(in_dim):
    def body(h, a, w1, b1, w2, b2, g, bt, out):
        z = h[...] + a[...]
        out[...] = _ln_relu_mlp(z, w1[...], b1[...], w2[...], b2[...],
                                g[...], bt[...])

    blk = lambda dd: pl.BlockSpec((BN, dd), lambda i: (i, 0))
    full = lambda shape: pl.BlockSpec(shape, lambda i: (0, 0))
    in_specs = [blk(in_dim), blk(in_dim),
                full((in_dim, 512)), full((1, 512)), full((512, 512)),
                full((1, 512)), full((1, 512)), full((1, 512))]
    return pl.pallas_call(
        body, grid=(N // BN,), in_specs=in_specs,
        out_specs=blk(512),
        out_shape=jax.ShapeDtypeStruct((N, 512), jnp.float32),
        name="gin_mlp")


@functools.lru_cache(maxsize=None)
def _mlp_final():
    def body(h, a, w1, b1, w2, b2, g, bt, wc1, bc1, wc2r, bc2, out):
        z = h[...] + a[...]
        hn = _ln_relu_mlp(z, w1[...], b1[...], w2[...], b2[...], g[...],
                          bt[...])
        u = jnp.maximum(jnp.dot(hn, wc1[...], preferred_element_type=jnp.float32)
                        + bc1[...], 0.0)
        out[...] = (jnp.sum(u * wc2r[...], axis=-1, keepdims=True) + bc2[...])

    blk = lambda dd: pl.BlockSpec((BN, dd), lambda i: (i, 0))
    full = lambda shape: pl.BlockSpec(shape, lambda i: (0, 0))
    in_specs = [blk(512), blk(512),
                full((512, 512)), full((1, 512)), full((512, 512)),
                full((1, 512)), full((1, 512)), full((1, 512)),
                full((512, 512)), full((1, 512)), full((1, 512)),
                full((1, 1))]
    return pl.pallas_call(
        body, grid=(N // BN,), in_specs=in_specs,
        out_specs=pl.BlockSpec((BN, 1), lambda i: (i, 0)),
        out_shape=jax.ShapeDtypeStruct((N, 1), jnp.float32),
        name="gin_mlp_final")


def kernel(x, edge_index, params):
    pad = EPAD - E
    src_p = jnp.concatenate([edge_index[0], jnp.zeros((pad,), jnp.int32)])
    dst_p = jnp.concatenate([edge_index[1], jnp.full((pad,), N, jnp.int32)])
    bsrc, bdst = _partition()(src_p, dst_p)
    bsrc = bsrc.reshape(NBKT, BKT_ROWS, 128)
    bdst = bdst.reshape(NBKT, BKT_ROWS, 128)

    h = x
    out = None
    for li, p in enumerate(params["layers"]):
        d = h.shape[1]
        zeros = jnp.zeros((ACC, d // 128, 128), jnp.float32)
        agg = _segsum(d)(h.reshape(N, d // 128, 128), bsrc, bdst, zeros)
        agg = agg.reshape(N, d)
        w1 = p["W1"]
        b1 = p["b1"].reshape(1, -1)
        w2 = p["W2"]
        b2 = p["b2"].reshape(1, -1)
        g = p["gamma"].reshape(1, -1)
        bt = p["beta"].reshape(1, -1)
        if li < 3:
            h = _mlp_hidden(d)(h, agg, w1, b1, w2, b2, g, bt)
        else:
            out = _mlp_final()(
                h, agg, w1, b1, w2, b2, g, bt,
                params["Wc1"], params["bc1"].reshape(1, -1),
                params["Wc2"].reshape(1, -1), params["bc2"].reshape(1, 1))
    return out


# speedup vs baseline: 7.1740x; 7.1740x over previous
import functools

import jax
import jax.numpy as jnp
from jax import lax
from jax.experimental import pallas as pl
from jax.experimental.pallas import tpu as pltpu
from jax.experimental.pallas import tpu_sc as plsc

N = 10000
E = 160000
F = 128
EROWS = 1280
EPAD = EROWS * F
NSUB = 16
ROWS_PER_TILE = EROWS // NSUB
NJ = N + 16
BN = 1000
GRP = 8
GROUPS = ROWS_PER_TILE // GRP
NSPLIT = 624


@functools.lru_cache(maxsize=None)
def _segsum(nchunks):
    mesh = plsc.VectorSubcoreMesh(core_axis_name="c", subcore_axis_name="s",
                                  num_cores=2, num_subcores=NSUB)
    out_type = tuple(jax.ShapeDtypeStruct((N, F), jnp.float32)
                     for _ in range(nchunks))
    scratch = [
        pltpu.VMEM((2, GRP, F), jnp.int32),
        pltpu.VMEM((2, GRP, F), jnp.int32),
        pltpu.VMEM((2, F, F), jnp.float32),
        pltpu.VMEM_SHARED((NJ, F), jnp.float32),
        pltpu.SemaphoreType.DMA,
        pltpu.SemaphoreType.DMA,
        pltpu.SemaphoreType.DMA,
        pltpu.SemaphoreType.DMA,
        pltpu.SemaphoreType.DMA,
    ]

    def body(*refs):
        hs = refs[:nchunks]
        src_r, dst_r, zeros_r = refs[nchunks:nchunks + 3]
        aggs = refs[nchunks + 3:2 * nchunks + 3]
        (srcb, dstb, gbuf, accum,
         sem0, sem1, ssem0, ssem1, semi) = refs[2 * nchunks + 3:]
        gsem = (sem0, sem1)
        ssem = (ssem0, ssem1)
        c = lax.axis_index("c")
        s = lax.axis_index("s")
        base = s * ROWS_PER_TILE

        def grp_slice(g):
            return pl.ds(pl.multiple_of(base + g * GRP, 8), GRP)

        def run_chunk(h_ref, agg_ref):
            zoff = pl.multiple_of(s * NSPLIT, 8)
            pltpu.sync_copy(zeros_r.at[pl.ds(zoff, NSPLIT)],
                            accum.at[pl.ds(zoff, NSPLIT)])

            @pl.when(s == 0)
            def _():
                tail = pl.ds(NSUB * NSPLIT, NJ - NSUB * NSPLIT)
                pltpu.sync_copy(zeros_r.at[tail], accum.at[tail])

            plsc.subcore_barrier()

            def scat_wait(b):
                pltpu.make_async_copy(gbuf.at[b],
                                      accum.at[dstb.at[0].at[0]],
                                      ssem[b]).wait()

            pltpu.sync_copy(src_r.at[grp_slice(0)], srcb.at[0])
            pltpu.sync_copy(dst_r.at[grp_slice(0)], dstb.at[0])
            pltpu.async_copy(h_ref.at[srcb.at[0].at[0]], gbuf.at[0], sem0)

            def step(g, carry):
                p = g % 2

                @pl.when(g > 0)
                def _():
                    scat_wait(0)
                    scat_wait(1)
                    pltpu.make_async_copy(src_r.at[grp_slice(g)], srcb.at[p],
                                          semi).wait()
                    pltpu.make_async_copy(dst_r.at[grp_slice(g)], dstb.at[p],
                                          semi).wait()
                    pltpu.async_copy(h_ref.at[srcb.at[p].at[0]], gbuf.at[0],
                                     sem0)

                @pl.when(g < GROUPS - 1)
                def _():
                    pltpu.async_copy(src_r.at[grp_slice(g + 1)],
                                     srcb.at[1 - p], semi)
                    pltpu.async_copy(dst_r.at[grp_slice(g + 1)],
                                     dstb.at[1 - p], semi)

                for j in range(GRP):
                    bj = j % 2
                    if j < GRP - 1:
                        if j > 0:
                            scat_wait(1 - bj)
                        pltpu.async_copy(h_ref.at[srcb.at[p].at[j + 1]],
                                         gbuf.at[1 - bj], gsem[1 - bj])
                    pltpu.make_async_copy(h_ref.at[srcb.at[p].at[j]],
                                          gbuf.at[bj], gsem[bj]).wait()
                    pltpu.async_copy(gbuf.at[bj],
                                     accum.at[dstb.at[p].at[j]],
                                     ssem[bj], add=True)
                return carry

            lax.fori_loop(0, GROUPS, step, 0)
            scat_wait(0)
            scat_wait(1)
            plsc.subcore_barrier()
            ooff = pl.multiple_of(s * NSPLIT, 8)
            pltpu.sync_copy(accum.at[pl.ds(ooff, NSPLIT)],
                            agg_ref.at[pl.ds(ooff, NSPLIT)])

            @pl.when(s == 0)
            def _():
                tail = pl.ds(NSUB * NSPLIT, N - NSUB * NSPLIT)
                pltpu.sync_copy(accum.at[tail], agg_ref.at[tail])

            plsc.subcore_barrier()

        half = nchunks // 2

        @pl.when(c == 0)
        def _():
            for q in range(half):
                run_chunk(hs[q], aggs[q])

        @pl.when(c == 1)
        def _():
            for q in range(half, nchunks):
                run_chunk(hs[q], aggs[q])

    return pl.kernel(body, out_type=out_type, mesh=mesh,
                     scratch_types=scratch, name=f"gin_segsum{nchunks}")


def _ln_relu_mlp(z, w1, b1, w2, b2, g, bt):
    t = jnp.maximum(jnp.dot(z, w1, preferred_element_type=jnp.float32) + b1, 0.0)
    t = jnp.dot(t, w2, preferred_element_type=jnp.float32) + b2
    mu = jnp.mean(t, axis=-1, keepdims=True)
    d = t - mu
    var = jnp.mean(d * d, axis=-1, keepdims=True)
    t = d * lax.rsqrt(var + 1e-5) * g + bt
    return jnp.maximum(t, 0.0)


@functools.lru_cache(maxsize=None)
def _mlp_hidden(nc_in, in_dim):
    def body(*refs):
        hs = refs[:nc_in]
        ags = refs[nc_in:2 * nc_in]
        w1, b1, w2, b2, g, bt = refs[2 * nc_in:2 * nc_in + 6]
        outs = refs[2 * nc_in + 6:]
        z = jnp.concatenate([hs[i][...] + ags[i][...] for i in range(nc_in)],
                            axis=-1)
        hn = _ln_relu_mlp(z, w1[...], b1[...], w2[...], b2[...], g[...], bt[...])
        for q in range(4):
            outs[q][...] = hn[:, q * F:(q + 1) * F]

    blk = pl.BlockSpec((BN, F), lambda i: (i, 0))
    full = lambda shape: pl.BlockSpec(shape, lambda i: (0, 0))
    in_specs = ([blk] * (2 * nc_in)
                + [full((in_dim, 512)), full((1, 512)), full((512, 512)),
                   full((1, 512)), full((1, 512)), full((1, 512))])
    return pl.pallas_call(
        body,
        grid=(N // BN,),
        in_specs=in_specs,
        out_specs=[blk] * 4,
        out_shape=[jax.ShapeDtypeStruct((N, F), jnp.float32)] * 4,
        name="gin_mlp",
    )


@functools.lru_cache(maxsize=None)
def _mlp_final(nc_in):
    def body(*refs):
        hs = refs[:nc_in]
        ags = refs[nc_in:2 * nc_in]
        w1, b1, w2, b2, g, bt, wc1, bc1, wc2r, bc2 = refs[2 * nc_in:2 * nc_in + 10]
        out, = refs[2 * nc_in + 10:]
        z = jnp.concatenate([hs[i][...] + ags[i][...] for i in range(nc_in)],
                            axis=-1)
        hn = _ln_relu_mlp(z, w1[...], b1[...], w2[...], b2[...], g[...],
                          bt[...])
        u = jnp.maximum(jnp.dot(hn, wc1[...], preferred_element_type=jnp.float32)
                        + bc1[...], 0.0)
        out[...] = (jnp.sum(u * wc2r[...], axis=-1, keepdims=True) + bc2[...])

    blk = pl.BlockSpec((BN, F), lambda i: (i, 0))
    full = lambda shape: pl.BlockSpec(shape, lambda i: (0, 0))
    in_specs = ([blk] * (2 * nc_in)
                + [full((512, 512)), full((1, 512)), full((512, 512)),
                   full((1, 512)), full((1, 512)), full((1, 512)),
                   full((512, 512)), full((1, 512)), full((1, 512)),
                   full((1, 1))])
    return pl.pallas_call(
        body,
        grid=(N // BN,),
        in_specs=in_specs,
        out_specs=pl.BlockSpec((BN, 1), lambda i: (i, 0)),
        out_shape=jax.ShapeDtypeStruct((N, 1), jnp.float32),
        name="gin_mlp_final",
    )


def kernel(x, edge_index, params):
    src = edge_index[0]
    dst = edge_index[1]
    pad = EPAD - E
    src_p = jnp.concatenate([src, jnp.zeros((pad,), jnp.int32)])
    src_p = src_p.reshape(EROWS, F)
    dst_p = jnp.concatenate([dst, jnp.full((pad,), N, jnp.int32)])
    dst_p = dst_p.reshape(EROWS, F)
    zeros = jnp.zeros((NJ, F), jnp.float32)

    hs = [x[:, :F], x[:, F:]]
    out = None
    for li, p in enumerate(params["layers"]):
        nc = len(hs)
        aggs = _segsum(nc)(*hs, src_p, dst_p, zeros)
        w1 = p["W1"]
        b1 = p["b1"].reshape(1, -1)
        w2 = p["W2"]
        b2 = p["b2"].reshape(1, -1)
        g = p["gamma"].reshape(1, -1)
        bt = p["beta"].reshape(1, -1)
        if li < 3:
            hs = list(_mlp_hidden(nc, w1.shape[0])(*hs, *aggs, w1, b1, w2, b2,
                                                   g, bt))
        else:
            out = _mlp_final(nc)(
                *hs, *aggs, w1, b1, w2, b2, g, bt,
                params["Wc1"], params["bc1"].reshape(1, -1),
                params["Wc2"].reshape(1, -1), params["bc2"].reshape(1, 1))
    return out
